# Initial kernel scaffold; baseline (speedup 1.0000x reference)
import jax
import jax.numpy as jnp
from jax.experimental import pallas as pl


def kernel(events, features, offsets):
    raise NotImplementedError("write your pallas kernel here")


# speedup vs baseline: 1.5073x; 1.5073x over previous
import jax
import jax.numpy as jnp
from jax import lax
from jax.experimental import pallas as pl
from jax.experimental.pallas import tpu as pltpu
from jax.experimental.pallas import tpu_sc as plsc

H = 256
W = 256
DIM = 32
B = 8
N = 524288

NC = 2
NS = 16
L = 16
NW = NC * NS

EPT = N // NW
A_SB = 4
A_CH = EPT // A_SB

CHUNK = 32768
NCHUNK = B * H * W // CHUNK
NPASS = NCHUNK // NC
DUMP = CHUNK
SROWS = CHUNK + 8

G = 128
FSZ = EPT + G + 8
TRASH = EPT + G
SENT = EPT

CPT = CHUNK // NS
OSB = 256
NOSB = CPT // OSB
ZR = 128

RNE = 8388608.0


def _body(ev_hbm, feat_hbm, off_hbm, out_hbm,
          off_buf, lin_buf, filt_ids,
          id_buf, cell_buf, feat_buf, ones_buf,
          s_buf, c_buf, inv_buf, plane_buf, zsum, zcnt,
          sums_sh, cnts_sh, gsem, psem):
    cid = lax.axis_index("c")
    sid = lax.axis_index("s")
    wid = sid * NC + cid
    iota = lax.iota(jnp.int32, L)

    pltpu.sync_copy(off_hbm, off_buf.at[pl.ds(0, B)])
    off_vec = off_buf[pl.ds(0, L)]

    def fill_ones(k, _):
        ones_buf[pl.ds(k * L, L)] = jnp.ones((L,), jnp.float32)
        return 0
    lax.fori_loop(0, G // L, fill_ones, 0)

    def fill_zsum(k, _):
        r = k // (DIM // L)
        col = (k % (DIM // L)) * L
        zsum[r, pl.ds(col, L)] = jnp.zeros((L,), jnp.float32)
        return 0
    lax.fori_loop(0, ZR * (DIM // L), fill_zsum, 0)

    def fill_zcnt(k, _):
        zcnt[pl.ds(k * L, L)] = jnp.zeros((L,), jnp.float32)
        return 0
    lax.fori_loop(0, ZR // L, fill_zcnt, 0)

    lin_buf[pl.ds(SENT, L)] = jnp.full((L,), jnp.int32(0x7FFFFFF), jnp.int32)

    def phase_a_sb(sb, _):
        pltpu.sync_copy(ev_hbm.at[pl.ds((wid * EPT + sb * A_CH) * 2, A_CH * 2)],
                        filt_ids.at[pl.ds(0, A_CH * 2)])

        def body(i, _):
            rows = i * L + iota
            xg = plsc.bitcast(plsc.load_gather(filt_ids, [rows * 2]), jnp.float32)
            yg = plsc.bitcast(plsc.load_gather(filt_ids, [rows * 2 + 1]), jnp.float32)
            xr = (xg * float(W) + RNE) - RNE
            yr = (yg * float(H) + RNE) - RNE
            xi = jnp.minimum(jnp.maximum(xr, 0.0), float(W - 1)).astype(jnp.int32)
            yi = jnp.minimum(jnp.maximum(yr, 0.0), float(H - 1)).astype(jnp.int32)
            j = wid * EPT + sb * A_CH + i * L + iota
            b = jnp.zeros((L,), jnp.int32)
            for k in range(B - 1):
                b = b + jnp.where(j >= off_vec[k], 1, 0).astype(jnp.int32)
            lin = b * (H * W) + yi * W + xi
            lin_buf[pl.ds(sb * A_CH + i * L, L)] = lin
            return 0
        lax.fori_loop(0, A_CH // L, body, 0)
        return 0
    lax.fori_loop(0, A_SB, phase_a_sb, 0)

    def zero_sub(q, _):
        cell0 = sid * CPT + q * ZR
        pltpu.sync_copy(zsum, sums_sh.at[pl.ds(cell0, ZR)])
        pltpu.sync_copy(zcnt, cnts_sh.at[pl.ds(cell0, ZR)])
        return 0
    lax.fori_loop(0, CPT // ZR, zero_sub, 0)
    plsc.subcore_barrier()

    def do_pass(p, _):
        chunk_id = cid * NPASS + p

        def filt(i, c):
            lin = lin_buf[pl.ds(i * L, L)]
            m = lax.shift_right_logical(lin, 15) == chunk_id
            s = plsc.cumsum(m.astype(jnp.int32))
            pos = jnp.where(m, c + s - 1, TRASH)
            plsc.store_scatter(filt_ids, [pos], i * L + iota)
            return c + s[L - 1]
        c = lax.fori_loop(0, EPT // L, filt, jnp.int32(0))

        def pad(k, _):
            filt_ids[pl.ds(c + k * L, L)] = jnp.full((L,), SENT, jnp.int32)
            return 0
        lax.fori_loop(0, G // L, pad, 0)

        n_g = lax.shift_right_logical(c + (G - 1), 7)

        def group(g, _):
            base = g * G

            def cp(k, _):
                loc = filt_ids[pl.ds(base + k * L, L)]
                lin = plsc.load_gather(lin_buf, [loc])
                m = lax.shift_right_logical(lin, 15) == chunk_id
                cell = jnp.where(m, lin & (CHUNK - 1), jnp.int32(DUMP))
                gid = jnp.minimum(wid * EPT + loc, jnp.int32(N - 1))
                id_buf[pl.ds(k * L, L)] = gid
                cell_buf[pl.ds(k * L, L)] = cell
                return 0
            lax.fori_loop(0, G // L, cp, 0)
            pltpu.async_copy(feat_hbm.at[id_buf], feat_buf, gsem).wait()
            pltpu.sync_copy(feat_buf, sums_sh.at[cell_buf], add=True)
            pltpu.sync_copy(ones_buf, cnts_sh.at[cell_buf], add=True)
            return 0
        lax.fori_loop(0, n_g, group, 0)
        plsc.subcore_barrier()

        b_idx = lax.shift_right_logical(chunk_id, 1)
        yh = chunk_id & 1

        def out_sub(sub, _):
            cell0 = sid * CPT + sub * OSB
            pltpu.sync_copy(sums_sh.at[pl.ds(cell0, OSB)], s_buf)
            pltpu.sync_copy(cnts_sh.at[pl.ds(cell0, OSB)], c_buf)

            def inv_k(k, _):
                cv = c_buf[pl.ds(k * L, L)]
                inv_buf[pl.ds(k * L, L)] = 1.0 / jnp.maximum(cv, 1.0)
                return 0
            lax.fori_loop(0, OSB // L, inv_k, 0)

            out0 = (b_idx * DIM * H * W + yh * (CHUNK // 2)
                    + sid * CPT + sub * OSB)

            def per_d(d, _):
                def tr(k, _):
                    rows = k * L + iota
                    v = plsc.load_gather(s_buf, [rows, jnp.full((L,), d, jnp.int32)])
                    v = v * inv_buf[pl.ds(k * L, L)]
                    plane_buf[d, pl.ds(k * L, L)] = v
                    return 0
                lax.fori_loop(0, OSB // L, tr, 0)
                off = out0 + d * (H * W)
                pltpu.async_copy(plane_buf.at[d], out_hbm.at[pl.ds(off, OSB)], psem)
                return 0
            lax.fori_loop(0, DIM, per_d, 0)

            def drain(d, _):
                off = out0 + d * (H * W)
                pltpu.make_async_copy(plane_buf.at[d], out_hbm.at[pl.ds(off, OSB)], psem).wait()
                return 0
            lax.fori_loop(0, DIM, drain, 0)

            def rezero(q, _):
                pltpu.sync_copy(zsum, sums_sh.at[pl.ds(cell0 + q * ZR, ZR)])
                pltpu.sync_copy(zcnt, cnts_sh.at[pl.ds(cell0 + q * ZR, ZR)])
                return 0
            lax.fori_loop(0, OSB // ZR, rezero, 0)
            return 0
        lax.fori_loop(0, NOSB, out_sub, 0)
        plsc.subcore_barrier()
        return 0
    lax.fori_loop(0, NPASS, do_pass, 0)


def kernel(events, features, offsets):
    mesh = plsc.VectorSubcoreMesh(core_axis_name="c", subcore_axis_name="s",
                                  num_cores=NC, num_subcores=NS)
    run = pl.kernel(
        _body,
        out_type=jax.ShapeDtypeStruct((B * DIM * H * W,), jnp.float32),
        mesh=mesh,
        scratch_types=[
            pltpu.VMEM((L,), jnp.int32),
            pltpu.VMEM((EPT + L,), jnp.int32),
            pltpu.VMEM((FSZ,), jnp.int32),
            pltpu.VMEM((G,), jnp.int32),
            pltpu.VMEM((G,), jnp.int32),
            pltpu.VMEM((G, DIM), jnp.float32),
            pltpu.VMEM((G,), jnp.float32),
            pltpu.VMEM((OSB, DIM), jnp.float32),
            pltpu.VMEM((OSB,), jnp.float32),
            pltpu.VMEM((OSB,), jnp.float32),
            pltpu.VMEM((DIM, OSB), jnp.float32),
            pltpu.VMEM((ZR, DIM), jnp.float32),
            pltpu.VMEM((ZR,), jnp.float32),
            pltpu.VMEM_SHARED((SROWS, DIM), jnp.float32),
            pltpu.VMEM_SHARED((SROWS,), jnp.float32),
            pltpu.SemaphoreType.DMA,
            pltpu.SemaphoreType.DMA,
        ],
        compiler_params=pltpu.CompilerParams(
            use_tc_tiling_on_sc=False, needs_layout_passes=False),
    )
    ev_flat = jax.lax.bitcast_convert_type(events, jnp.int32).reshape(N * 2)
    out = run(ev_flat, features, offsets)
    return out.reshape(B, DIM, H, W)
